# baseline (speedup 1.0000x reference)
import jax
import jax.numpy as jnp
from jax import lax
from jax.experimental import pallas as pl
from jax.experimental.pallas import tpu as pltpu


def _make_kernel(L, W, NL, FF):
    def _fused_kernel(
        seq_ref, x_ref, w2_ref, emb_ref, wse_ref, ffb_ref, netw_ref,
        netb_ref, outw_ref, outb_ref, tgt_ref, out_ref, loss_ref,
        g_ref, wmb_ref,
    ):
        b = pl.program_id(0)

        @pl.when(b == 0)
        def _init():
            loss_ref[0, 0] = 0.0

        g_ref[0:L, :] = jnp.dot(
            x_ref[0], w2_ref[...], preferred_element_type=jnp.float32
        )
        g_ref[L : L + 16, :] = jnp.broadcast_to(
            g_ref[L - 1 : L, :], (16, 2 * FF)
        )
        wmb_ref[...] = (
            jnp.dot(emb_ref[...], wse_ref[...],
                    preferred_element_type=jnp.float32)
            + ffb_ref[...]
        )

        netw = netw_ref[...]
        netb = netb_ref[...]
        outw = outw_ref[...]
        outb = outb_ref[...]
        seqlen = seq_ref[0, 0, 0]
        row = lax.broadcasted_iota(jnp.int32, (L, 1), 0)

        bg = g_ref[0:L, 0:FF]
        lgs = []
        msk = []
        for w in range(W):
            eg = g_ref[w : w + L, FF : 2 * FF]
            h = jnp.maximum(bg + eg + wmb_ref[w : w + 1, :], 0.0)
            h = jnp.maximum(
                jnp.dot(h, netw, preferred_element_type=jnp.float32) + netb,
                0.0,
            )
            lg = jnp.dot(h, outw, preferred_element_type=jnp.float32) + outb
            lgs.append(lg)
            m = (row + w < seqlen).astype(jnp.float32)
            msk.append(jnp.broadcast_to(m, (L, NL)))
        cat = jnp.concatenate(lgs, axis=1)
        mcat = jnp.concatenate(msk, axis=1)
        out_ref[0] = cat
        z = tgt_ref[0]
        bce = (
            jnp.maximum(cat, 0.0)
            - cat * z
            + jnp.log1p(jnp.exp(-jnp.abs(cat)))
        )
        loss_ref[0, 0] += jnp.sum(bce * mcat)

    return _fused_kernel


def kernel(inputs, sequence_lengths, span_targets, embed_table, ff_W, ff_b,
           net_W, net_b, out_W, out_b):
    B, L, D = inputs.shape
    W, SE = embed_table.shape
    FF = ff_W.shape[1]
    NET = net_W.shape[1]
    NL = out_W.shape[1]

    w2 = jnp.concatenate([ff_W[:D], ff_W[D : 2 * D]], axis=1)
    wse = ff_W[2 * D :]
    ffb2 = ff_b.reshape(1, FF)
    netb2 = net_b.reshape(1, NET)
    outb2 = out_b.reshape(1, NL)
    seq2 = sequence_lengths.reshape(B, 1, 1).astype(jnp.int32)
    tgt2 = span_targets.reshape(B, L, W * NL)

    logits_flat, loss = pl.pallas_call(
        _make_kernel(L, W, NL, FF),
        grid=(B,),
        in_specs=[
            pl.BlockSpec((1, 1, 1), lambda b: (b, 0, 0),
                         memory_space=pltpu.SMEM),
            pl.BlockSpec((1, L, D), lambda b: (b, 0, 0)),
            pl.BlockSpec((D, 2 * FF), lambda b: (0, 0)),
            pl.BlockSpec((W, SE), lambda b: (0, 0)),
            pl.BlockSpec((SE, FF), lambda b: (0, 0)),
            pl.BlockSpec((1, FF), lambda b: (0, 0)),
            pl.BlockSpec((FF, NET), lambda b: (0, 0)),
            pl.BlockSpec((1, NET), lambda b: (0, 0)),
            pl.BlockSpec((NET, NL), lambda b: (0, 0)),
            pl.BlockSpec((1, NL), lambda b: (0, 0)),
            pl.BlockSpec((1, L, W * NL), lambda b: (b, 0, 0)),
        ],
        out_specs=[
            pl.BlockSpec((1, L, W * NL), lambda b: (b, 0, 0)),
            pl.BlockSpec((1, 1), lambda b: (0, 0),
                         memory_space=pltpu.SMEM),
        ],
        out_shape=[
            jax.ShapeDtypeStruct((B, L, W * NL), jnp.float32),
            jax.ShapeDtypeStruct((1, 1), jnp.float32),
        ],
        scratch_shapes=[
            pltpu.VMEM((L + 16, 2 * FF), jnp.float32),
            pltpu.VMEM((W, FF), jnp.float32),
        ],
    )(seq2, inputs, w2, embed_table, wse, ffb2, net_W, netb2, out_W, outb2,
      tgt2)

    logits = logits_flat.reshape(B, L, W, NL)
    return logits, loss[0, 0]


# speedup vs baseline: 1.0483x; 1.0483x over previous
import jax
import jax.numpy as jnp
from jax import lax
from jax.experimental import pallas as pl
from jax.experimental.pallas import tpu as pltpu


def _make_kernel(L, W, NL, FF):
    def _fused_kernel(
        seq_ref, x_ref, fw_ref, emb_ref, ffb_ref, netw_ref,
        netb_ref, outw_ref, outb_ref, tgt_ref, out_ref, loss_ref,
        g_ref, wmb_ref,
    ):
        b = pl.program_id(0)
        D = x_ref.shape[2]

        @pl.when(b == 0)
        def _init():
            loss_ref[0, 0] = 0.0

        x = x_ref[0]
        g_ref[0:L, 0:FF] = jnp.dot(
            x, fw_ref[0:D, :], preferred_element_type=jnp.float32
        )
        g_ref[0:L, FF : 2 * FF] = jnp.dot(
            x, fw_ref[D : 2 * D, :], preferred_element_type=jnp.float32
        )
        g_ref[L : L + 16, :] = jnp.broadcast_to(
            g_ref[L - 1 : L, :], (16, 2 * FF)
        )
        wmb_ref[...] = (
            jnp.dot(emb_ref[...], fw_ref[2 * D :, :],
                    preferred_element_type=jnp.float32)
            + ffb_ref[...]
        )

        netw = netw_ref[...]
        netb = netb_ref[...]
        outw = outw_ref[...]
        outb = outb_ref[...]
        seqlen = seq_ref[0, 0, 0]
        row = lax.broadcasted_iota(jnp.int32, (L, 1), 0)

        bg = g_ref[0:L, 0:FF]
        lgs = []
        msk = []
        for w in range(W):
            eg = g_ref[w : w + L, FF : 2 * FF]
            h = jnp.maximum(bg + eg + wmb_ref[w : w + 1, :], 0.0)
            h = jnp.maximum(
                jnp.dot(h, netw, preferred_element_type=jnp.float32) + netb,
                0.0,
            )
            lg = jnp.dot(h, outw, preferred_element_type=jnp.float32) + outb
            lgs.append(lg)
            m = (row + w < seqlen).astype(jnp.float32)
            msk.append(jnp.broadcast_to(m, (L, NL)))
        cat = jnp.concatenate(lgs, axis=1)
        mcat = jnp.concatenate(msk, axis=1)
        out_ref[0] = cat
        z = tgt_ref[0]
        bce = (
            jnp.maximum(cat, 0.0)
            - cat * z
            + jnp.log1p(jnp.exp(-jnp.abs(cat)))
        )
        loss_ref[0, 0] += jnp.sum(bce * mcat)

    return _fused_kernel


def kernel(inputs, sequence_lengths, span_targets, embed_table, ff_W, ff_b,
           net_W, net_b, out_W, out_b):
    B, L, D = inputs.shape
    W, SE = embed_table.shape
    FF = ff_W.shape[1]
    NET = net_W.shape[1]
    NL = out_W.shape[1]

    ffb2 = ff_b.reshape(1, FF)
    netb2 = net_b.reshape(1, NET)
    outb2 = out_b.reshape(1, NL)
    seq2 = sequence_lengths.reshape(B, 1, 1).astype(jnp.int32)
    tgt2 = span_targets.reshape(B, L, W * NL)

    logits_flat, loss = pl.pallas_call(
        _make_kernel(L, W, NL, FF),
        grid=(B,),
        in_specs=[
            pl.BlockSpec((1, 1, 1), lambda b: (b, 0, 0),
                         memory_space=pltpu.SMEM),
            pl.BlockSpec((1, L, D), lambda b: (b, 0, 0)),
            pl.BlockSpec((2 * D + SE, FF), lambda b: (0, 0)),
            pl.BlockSpec((W, SE), lambda b: (0, 0)),
            pl.BlockSpec((1, FF), lambda b: (0, 0)),
            pl.BlockSpec((FF, NET), lambda b: (0, 0)),
            pl.BlockSpec((1, NET), lambda b: (0, 0)),
            pl.BlockSpec((NET, NL), lambda b: (0, 0)),
            pl.BlockSpec((1, NL), lambda b: (0, 0)),
            pl.BlockSpec((1, L, W * NL), lambda b: (b, 0, 0)),
        ],
        out_specs=[
            pl.BlockSpec((1, L, W * NL), lambda b: (b, 0, 0)),
            pl.BlockSpec((1, 1), lambda b: (0, 0),
                         memory_space=pltpu.SMEM),
        ],
        out_shape=[
            jax.ShapeDtypeStruct((B, L, W * NL), jnp.float32),
            jax.ShapeDtypeStruct((1, 1), jnp.float32),
        ],
        scratch_shapes=[
            pltpu.VMEM((L + 16, 2 * FF), jnp.float32),
            pltpu.VMEM((W, FF), jnp.float32),
        ],
    )(seq2, inputs, ff_W, embed_table, ffb2, net_W, netb2, out_W, outb2,
      tgt2)

    logits = logits_flat.reshape(B, L, W, NL)
    return logits, loss[0, 0]
